# Initial kernel scaffold; baseline (speedup 1.0000x reference)
import jax
import jax.numpy as jnp
from jax.experimental import pallas as pl
from jax.experimental.pallas import tpu as pltpu

MB = 1024 * 1024
BF16 = jnp.bfloat16


def _conv1_kernel(xl_ref, w_ref, b_ref, o_ref):
    best = None
    for a in range(2):
        for b in range(2):
            acc = None
            for di in range(2):
                for dj in range(2):
                    g = (a * 2 + b) * 4 + di * 2 + dj
                    p = xl_ref[0, di:di + 55, dj:dj + 56, :].reshape(3080, 48)
                    d = jax.lax.dot_general(
                        p, w_ref[g], (((1,), (0,)), ((), ())),
                        preferred_element_type=jnp.float32)
                    acc = d if acc is None else acc + d
            best = acc if best is None else jnp.maximum(best, acc)
    z = jnp.maximum(best + b_ref[...], 0.0)
    o_ref[0] = z.reshape(55, 56, 64).astype(o_ref.dtype)


def _u_range(a, di):
    lo = max(0, 2 * a - 4 * di)
    hi = min(3, 4 + 2 * a - 4 * di)
    return (lo, hi) if lo <= hi else None


def _conv2_kernel(zl_ref, w_ref, b_ref, o_ref):
    best = None
    for a in range(2):
        for b in range(2):
            acc = None
            for di in range(2):
                ur = _u_range(a, di)
                if ur is None:
                    continue
                u0, u1 = ur
                k0, k1 = u0 * 256, (u1 + 1) * 256
                for dj in range(2):
                    g = (a * 2 + b) * 4 + di * 2 + dj
                    p = zl_ref[0, di:di + 14, dj:dj + 16, k0:k1]
                    pm = p.reshape(224, k1 - k0)
                    d = jax.lax.dot_general(
                        pm, w_ref[g, k0:k1, :], (((1,), (0,)), ((), ())),
                        preferred_element_type=jnp.float32)
                    acc = d if acc is None else acc + d
            best = acc if best is None else jnp.maximum(best, acc)
    z = jnp.maximum(best + b_ref[...], 0.0)
    o_ref[0] = z.astype(o_ref.dtype)


def _fc_kernel(x_ref, w1_ref, b1_ref, w2_ref, b2_ref, o_ref):
    h = jax.lax.dot_general(
        x_ref[...], w1_ref[...], (((1,), (1,)), ((), ())),
        preferred_element_type=jnp.float32)
    h = jnp.maximum(h + b1_ref[...], 0.0)
    z = jax.lax.dot_general(
        h.astype(BF16), w2_ref[...], (((1,), (1,)), ((), ())),
        preferred_element_type=jnp.float32)
    z = jnp.maximum(z + b2_ref[...], 0.0)
    m = jnp.max(z, axis=-1, keepdims=True)
    e = jnp.exp(z - m)
    s = jnp.sum(e, axis=-1, keepdims=True)
    o_ref[...] = (z - m - jnp.log(s)).astype(o_ref.dtype)


def _pack_conv1_w(cw):
    z = jnp.zeros((3, 64), jnp.float32)
    mats = []
    for a in range(2):
        for b in range(2):
            for di in range(2):
                for dj in range(2):
                    rows = []
                    for pi in range(4):
                        for pj in range(4):
                            i = 4 * di + pi - 2 * a
                            j = 4 * dj + pj - 2 * b
                            if 0 <= i < 5 and 0 <= j < 5:
                                rows.append(cw[:, :, i, j].T)
                            else:
                                rows.append(z)
                    mats.append(jnp.concatenate(rows, axis=0))
    return jnp.stack(mats).astype(BF16)


def _pack_conv2_w(cw):
    z = jnp.zeros((64, 128), jnp.float32)
    mats = []
    for a in range(2):
        for b in range(2):
            for di in range(2):
                for dj in range(2):
                    blocks = []
                    for u in range(4):
                        for v in range(4):
                            i = 4 * di + u - 2 * a
                            j = 4 * dj + v - 2 * b
                            if 0 <= i < 5 and 0 <= j < 5:
                                blocks.append(cw[:, :, i, j].T)
                            else:
                                blocks.append(z)
                    mats.append(jnp.concatenate(blocks, axis=0))
    return jnp.stack(mats).astype(BF16)


def kernel(conv1_w, conv1_b, conv2_w, conv2_b, fc1_w, fc1_b, fc2_w, fc2_b, x):
    N = x.shape[0]

    xl = x.reshape(N, 3, 56, 4, 56, 4).transpose(0, 2, 4, 3, 5, 1)
    xl = xl.reshape(N, 56, 56, 48)
    xl = jnp.pad(xl, ((0, 0), (0, 0), (0, 8), (0, 0))).astype(BF16)

    w1g = _pack_conv1_w(conv1_w)
    b1 = conv1_b.reshape(1, 64)

    z = pl.pallas_call(
        _conv1_kernel,
        out_shape=jax.ShapeDtypeStruct((N, 55, 56, 64), BF16),
        grid_spec=pltpu.PrefetchScalarGridSpec(
            num_scalar_prefetch=0,
            grid=(N,),
            in_specs=[
                pl.BlockSpec((1, 56, 64, 48), lambda n: (n, 0, 0, 0)),
                pl.BlockSpec((16, 48, 64), lambda n: (0, 0, 0)),
                pl.BlockSpec((1, 64), lambda n: (0, 0)),
            ],
            out_specs=pl.BlockSpec((1, 55, 56, 64), lambda n: (n, 0, 0, 0)),
        ),
        compiler_params=pltpu.CompilerParams(
            dimension_semantics=("parallel",),
            vmem_limit_bytes=48 * MB),
    )(xl, w1g, b1)

    zp = jnp.pad(z[:, :, :55, :], ((0, 0), (0, 9), (0, 41), (0, 0)))
    zl = zp.reshape(N, 16, 4, 24, 4, 64).transpose(0, 1, 3, 2, 4, 5)
    zl = zl.reshape(N, 16, 24, 1024)

    w2g = _pack_conv2_w(conv2_w)
    b2 = conv2_b.reshape(1, 128)

    y = pl.pallas_call(
        _conv2_kernel,
        out_shape=jax.ShapeDtypeStruct((N, 224, 128), BF16),
        grid_spec=pltpu.PrefetchScalarGridSpec(
            num_scalar_prefetch=0,
            grid=(N,),
            in_specs=[
                pl.BlockSpec((1, 16, 24, 1024), lambda n: (n, 0, 0, 0)),
                pl.BlockSpec((16, 1024, 128), lambda n: (0, 0, 0)),
                pl.BlockSpec((1, 128), lambda n: (0, 0)),
            ],
            out_specs=pl.BlockSpec((1, 224, 128), lambda n: (n, 0, 0, 0)),
        ),
        compiler_params=pltpu.CompilerParams(
            dimension_semantics=("parallel",),
            vmem_limit_bytes=48 * MB),
    )(zl, w2g, b2)

    f2 = y.reshape(N, 14, 16, 128)[:, :13, :13, :]
    xf = f2.transpose(0, 3, 1, 2).reshape(N, 128 * 169).astype(BF16)

    return pl.pallas_call(
        _fc_kernel,
        out_shape=jax.ShapeDtypeStruct((N, 2), jnp.float32),
        grid_spec=pltpu.PrefetchScalarGridSpec(
            num_scalar_prefetch=0,
            grid=(1,),
            in_specs=[
                pl.BlockSpec((N, 128 * 169), lambda i: (0, 0)),
                pl.BlockSpec((128, 128 * 169), lambda i: (0, 0)),
                pl.BlockSpec((1, 128), lambda i: (0, 0)),
                pl.BlockSpec((2, 128), lambda i: (0, 0)),
                pl.BlockSpec((1, 2), lambda i: (0, 0)),
            ],
            out_specs=pl.BlockSpec((N, 2), lambda i: (0, 0)),
        ),
        compiler_params=pltpu.CompilerParams(
            dimension_semantics=("arbitrary",),
            vmem_limit_bytes=48 * MB),
    )(xf, fc1_w.astype(BF16), fc1_b.reshape(1, 128),
      fc2_w.astype(BF16), fc2_b.reshape(1, 2))


# speedup vs baseline: 25.4359x; 25.4359x over previous
import jax
import jax.numpy as jnp
from jax.experimental import pallas as pl
from jax.experimental.pallas import tpu as pltpu

MB = 1024 * 1024
BF16 = jnp.bfloat16


def _conv1_kernel(xl_ref, w_ref, b_ref, o_ref):
    best = None
    for a in range(2):
        for b in range(2):
            acc = None
            for di in range(2):
                for dj in range(2):
                    g = (a * 2 + b) * 4 + di * 2 + dj
                    p = xl_ref[0, di:di + 55, dj:dj + 56, :].reshape(3080, 48)
                    d = jax.lax.dot_general(
                        p, w_ref[g], (((1,), (0,)), ((), ())),
                        preferred_element_type=jnp.float32)
                    acc = d if acc is None else acc + d
            best = acc if best is None else jnp.maximum(best, acc)
    z = jnp.maximum(best + b_ref[...], 0.0)
    o_ref[0] = z.reshape(55, 56, 64).astype(o_ref.dtype)


def _u_range(a, di):
    lo = max(0, 2 * a - 4 * di)
    hi = min(3, 4 + 2 * a - 4 * di)
    return (lo, hi) if lo <= hi else None


def _conv2_kernel(zl_ref, w_ref, b_ref, o_ref):
    best = None
    for a in range(2):
        for b in range(2):
            acc = None
            for di in range(2):
                ur = _u_range(a, di)
                if ur is None:
                    continue
                u0, u1 = ur
                k0, k1 = u0 * 256, (u1 + 1) * 256
                for dj in range(2):
                    g = (a * 2 + b) * 4 + di * 2 + dj
                    p = zl_ref[0, di:di + 14, dj:dj + 16, k0:k1]
                    pm = p.reshape(224, k1 - k0)
                    d = jax.lax.dot_general(
                        pm, w_ref[g, k0:k1, :], (((1,), (0,)), ((), ())),
                        preferred_element_type=jnp.float32)
                    acc = d if acc is None else acc + d
            best = acc if best is None else jnp.maximum(best, acc)
    z = jnp.maximum(best + b_ref[...], 0.0)
    o_ref[0] = z.astype(o_ref.dtype)


def _fc_kernel(x_ref, w1_ref, b1_ref, w2_ref, b2_ref, o_ref):
    h = jax.lax.dot_general(
        x_ref[...], w1_ref[...], (((1,), (1,)), ((), ())),
        preferred_element_type=jnp.float32)
    h = jnp.maximum(h + b1_ref[...], 0.0)
    z = jax.lax.dot_general(
        h.astype(BF16), w2_ref[...], (((1,), (1,)), ((), ())),
        preferred_element_type=jnp.float32)
    z = jnp.maximum(z + b2_ref[...], 0.0)
    m = jnp.max(z, axis=-1, keepdims=True)
    e = jnp.exp(z - m)
    s = jnp.sum(e, axis=-1, keepdims=True)
    o_ref[...] = (z - m - jnp.log(s)).astype(o_ref.dtype)


def _pack_conv1_w(cw):
    z = jnp.zeros((3, 64), jnp.float32)
    mats = []
    for a in range(2):
        for b in range(2):
            for di in range(2):
                for dj in range(2):
                    rows = []
                    for pi in range(4):
                        for pj in range(4):
                            i = 4 * di + pi - 2 * a
                            j = 4 * dj + pj - 2 * b
                            if 0 <= i < 5 and 0 <= j < 5:
                                rows.append(cw[:, :, i, j].T)
                            else:
                                rows.append(z)
                    mats.append(jnp.concatenate(rows, axis=0))
    return jnp.stack(mats).astype(BF16)


def _pack_conv2_w(cw):
    z = jnp.zeros((64, 128), jnp.float32)
    mats = []
    for a in range(2):
        for b in range(2):
            for di in range(2):
                for dj in range(2):
                    blocks = []
                    for u in range(4):
                        for v in range(4):
                            i = 4 * di + u - 2 * a
                            j = 4 * dj + v - 2 * b
                            if 0 <= i < 5 and 0 <= j < 5:
                                blocks.append(cw[:, :, i, j].T)
                            else:
                                blocks.append(z)
                    mats.append(jnp.concatenate(blocks, axis=0))
    return jnp.stack(mats).astype(BF16)


def kernel(conv1_w, conv1_b, conv2_w, conv2_b, fc1_w, fc1_b, fc2_w, fc2_b, x):
    N = x.shape[0]

    xl = x.reshape(N, 3, 56, 4, 56, 4).transpose(0, 2, 4, 3, 5, 1)
    xl = xl.reshape(N, 56, 56, 48)
    xl = jnp.pad(xl, ((0, 0), (0, 0), (0, 8), (0, 0))).astype(BF16)

    w1g = _pack_conv1_w(conv1_w)
    b1 = conv1_b.reshape(1, 64)

    z = pl.pallas_call(
        _conv1_kernel,
        out_shape=jax.ShapeDtypeStruct((N, 55, 56, 64), BF16),
        grid_spec=pltpu.PrefetchScalarGridSpec(
            num_scalar_prefetch=0,
            grid=(N,),
            in_specs=[
                pl.BlockSpec((1, 56, 64, 48), lambda n: (n, 0, 0, 0)),
                pl.BlockSpec((16, 48, 64), lambda n: (0, 0, 0)),
                pl.BlockSpec((1, 64), lambda n: (0, 0)),
            ],
            out_specs=pl.BlockSpec((1, 55, 56, 64), lambda n: (n, 0, 0, 0)),
        ),
        compiler_params=pltpu.CompilerParams(
            dimension_semantics=("parallel",),
            vmem_limit_bytes=48 * MB),
    )(xl, w1g, b1)

    zp = jnp.pad(z[:, :, :55, :], ((0, 0), (0, 9), (0, 41), (0, 0)))
    zl = zp.reshape(N, 16, 4, 24, 4, 64).transpose(0, 1, 3, 2, 4, 5)
    zl = zl.reshape(N, 16, 24, 1024)

    w2g = _pack_conv2_w(conv2_w)
    b2 = conv2_b.reshape(1, 128)

    y = pl.pallas_call(
        _conv2_kernel,
        out_shape=jax.ShapeDtypeStruct((N, 224, 128), BF16),
        grid_spec=pltpu.PrefetchScalarGridSpec(
            num_scalar_prefetch=0,
            grid=(N,),
            in_specs=[
                pl.BlockSpec((1, 16, 24, 1024), lambda n: (n, 0, 0, 0)),
                pl.BlockSpec((16, 1024, 128), lambda n: (0, 0, 0)),
                pl.BlockSpec((1, 128), lambda n: (0, 0)),
            ],
            out_specs=pl.BlockSpec((1, 224, 128), lambda n: (n, 0, 0)),
        ),
        compiler_params=pltpu.CompilerParams(
            dimension_semantics=("parallel",),
            vmem_limit_bytes=48 * MB),
    )(zl, w2g, b2)

    f2 = y.reshape(N, 14, 16, 128)[:, :13, :13, :]
    xf = f2.transpose(0, 3, 1, 2).reshape(N, 128 * 169).astype(BF16)

    return pl.pallas_call(
        _fc_kernel,
        out_shape=jax.ShapeDtypeStruct((N, 2), jnp.float32),
        grid_spec=pltpu.PrefetchScalarGridSpec(
            num_scalar_prefetch=0,
            grid=(1,),
            in_specs=[
                pl.BlockSpec((N, 128 * 169), lambda i: (0, 0)),
                pl.BlockSpec((128, 128 * 169), lambda i: (0, 0)),
                pl.BlockSpec((1, 128), lambda i: (0, 0)),
                pl.BlockSpec((2, 128), lambda i: (0, 0)),
                pl.BlockSpec((1, 2), lambda i: (0, 0)),
            ],
            out_specs=pl.BlockSpec((N, 2), lambda i: (0, 0)),
        ),
        compiler_params=pltpu.CompilerParams(
            dimension_semantics=("arbitrary",),
            vmem_limit_bytes=48 * MB),
    )(xf, fc1_w.astype(BF16), fc1_b.reshape(1, 128),
      fc2_w.astype(BF16), fc2_b.reshape(1, 2))
